# baseline (speedup 1.0000x reference)
import functools

import jax
import jax.numpy as jnp
from jax import lax
from jax.experimental import pallas as pl
from jax.experimental.pallas import tpu as pltpu
from jax.experimental.pallas import tpu_sc as plsc

NUM_EXPERTS = 64
TOP_K = 8
BLOCK_T = 1024
TCH = 256
LPAD = NUM_EXPERTS + 1
NC, NS = 2, 16
NW = NC * NS


def _tc_block(x_ref, w_ref, logits_ref, probs_ref):
    logits = jnp.dot(x_ref[...], w_ref[...], preferred_element_type=jnp.float32)
    logits_ref[...] = logits
    m0 = jnp.max(logits, axis=1, keepdims=True)
    p = jnp.exp(logits - m0)
    probs_ref[...] = p / jnp.sum(p, axis=1, keepdims=True)


def _sc_routing(logits_hbm, ew_hbm, idx_hbm, lbuf, tdiag, tbuf, ewbuf, idxbuf,
                *, ch_per_w):
    wid = lax.axis_index("s") * NC + lax.axis_index("c")
    lane = lax.iota(jnp.int32, 16)

    def chunk_body(c, carry):
        g = wid * ch_per_w + c
        tb = g * TCH
        pltpu.sync_copy(logits_hbm.at[pl.ds(tb, TCH), :], lbuf)

        def zbody(i, carry2):
            for u in range(NUM_EXPERTS // 16):
                ewbuf[i, pl.ds(u * 16, 16)] = jnp.zeros((16,), jnp.float32)
            return carry2

        lax.fori_loop(0, TCH, zbody, 0)

        def gbody(lg, carry2):
            tok = lane + lg * 16
            for r in range(NUM_EXPERTS):
                tdiag[r, :] = plsc.load_gather(
                    lbuf, [tok, (lane + r) & (NUM_EXPERTS - 1)])
            for e in range(NUM_EXPERTS):
                tbuf[e, :] = plsc.load_gather(
                    tdiag, [(jnp.full((16,), e, jnp.int32) - lane)
                            & (NUM_EXPERTS - 1), lane])
            vals, idxs = [], []
            for _ in range(TOP_K):
                parts = []
                for c4 in range(4):
                    be = c4 * 16
                    rm = tbuf[be, :]
                    ri = jnp.full((16,), be, jnp.int32)
                    for e in range(be + 1, be + 16):
                        v = tbuf[e, :]
                        take = v > rm
                        rm = jnp.where(take, v, rm)
                        ri = jnp.where(take, jnp.full((16,), e, jnp.int32), ri)
                    parts.append((rm, ri))
                rm, ri = parts[0]
                for v, vi in parts[1:]:
                    take = v > rm
                    rm = jnp.where(take, v, rm)
                    ri = jnp.where(take, vi, ri)
                vals.append(rm)
                idxs.append(ri)
                plsc.store_scatter(tbuf, [ri, lane],
                                   jnp.full((16,), -1e30, jnp.float32))
            s = jnp.zeros((16,), jnp.float32)
            es = []
            for k in range(TOP_K):
                e_k = jnp.exp(vals[k] - vals[0])
                es.append(e_k)
                s = s + e_k
            rs = 1.0 / s
            for k in range(TOP_K):
                plsc.store_scatter(ewbuf, [tok, idxs[k]], es[k] * rs)
                plsc.store_scatter(idxbuf, [tok, jnp.full((16,), k, jnp.int32)],
                                   idxs[k])
            return carry2

        lax.fori_loop(0, TCH // 16, gbody, 0)
        pltpu.sync_copy(ewbuf, ew_hbm.at[pl.ds(tb, TCH), :])
        pltpu.sync_copy(idxbuf, idx_hbm.at[pl.ds(tb, TCH), :])
        return carry

    lax.fori_loop(0, ch_per_w, chunk_body, 0)


@jax.jit
def kernel(inputs, gate_kernel):
    n_tokens, d_model = inputs.shape
    grid = (n_tokens // BLOCK_T,)
    n_chunks = n_tokens // TCH
    tok_spec = lambda w: pl.BlockSpec((BLOCK_T, w), lambda i: (i, 0))
    logits, probs = pl.pallas_call(
        _tc_block,
        grid=grid,
        in_specs=[
            pl.BlockSpec((BLOCK_T, d_model), lambda i: (i, 0)),
            pl.BlockSpec((d_model, NUM_EXPERTS), lambda i: (0, 0)),
        ],
        out_specs=(tok_spec(NUM_EXPERTS), tok_spec(NUM_EXPERTS)),
        out_shape=(
            jax.ShapeDtypeStruct((n_tokens, NUM_EXPERTS), jnp.float32),
            jax.ShapeDtypeStruct((n_tokens, NUM_EXPERTS), jnp.float32),
        ),
        compiler_params=pltpu.CompilerParams(
            dimension_semantics=("arbitrary",),
        ),
    )(inputs, gate_kernel)

    ch_per_w = n_chunks // NW
    mesh = plsc.VectorSubcoreMesh(
        core_axis_name="c", subcore_axis_name="s",
        num_cores=NC, num_subcores=NS)
    ew, idx = pl.kernel(
        functools.partial(_sc_routing, ch_per_w=ch_per_w),
        out_type=(
            jax.ShapeDtypeStruct((n_tokens, NUM_EXPERTS), jnp.float32),
            jax.ShapeDtypeStruct((n_tokens, TOP_K), jnp.int32),
        ),
        mesh=mesh,
        compiler_params=pltpu.CompilerParams(needs_layout_passes=False),
        scratch_types=[
            pltpu.VMEM((TCH, NUM_EXPERTS), jnp.float32),
            pltpu.VMEM((NUM_EXPERTS, 16), jnp.float32),
            pltpu.VMEM((NUM_EXPERTS, 16), jnp.float32),
            pltpu.VMEM((TCH, NUM_EXPERTS), jnp.float32),
            pltpu.VMEM((TCH, TOP_K), jnp.int32),
        ],
    )(logits)
    return (ew, idx, logits, probs)


# speedup vs baseline: 1.1287x; 1.1287x over previous
import functools

import jax
import jax.numpy as jnp
from jax import lax
from jax.experimental import pallas as pl
from jax.experimental.pallas import tpu as pltpu
from jax.experimental.pallas import tpu_sc as plsc

NUM_EXPERTS = 64
TOP_K = 8
BLOCK_T = 1024
TCH = 256
NC, NS = 2, 16
NW = NC * NS


def _tc_block(x_ref, w_ref, logits_ref, probs_ref, lch_ref):
    logits = jnp.dot(x_ref[...], w_ref[...], preferred_element_type=jnp.float32)
    logits_ref[...] = logits
    m0 = jnp.max(logits, axis=1, keepdims=True)
    p = jnp.exp(logits - m0)
    probs_ref[...] = p / jnp.sum(p, axis=1, keepdims=True)
    nch = BLOCK_T // TCH
    lch_ref[...] = jnp.transpose(
        logits.reshape(nch, TCH, NUM_EXPERTS), (0, 2, 1))


def _sc_routing(lch_hbm, ew_hbm, idx_hbm, lbuf, ewbuf, idxbuf, *, ch_per_w):
    wid = lax.axis_index("s") * NC + lax.axis_index("c")
    lane = lax.iota(jnp.int32, 16)

    def chunk_body(c, carry):
        g = wid * ch_per_w + c
        pltpu.sync_copy(lch_hbm.at[g], lbuf)

        def zbody(i, carry2):
            for u in range(NUM_EXPERTS // 16):
                ewbuf[i, pl.ds(u * 16, 16)] = jnp.zeros((16,), jnp.float32)
            return carry2

        lax.fori_loop(0, TCH, zbody, 0)

        def gbody(lg, carry2):
            tok = lane + lg * 16
            vals, idxs = [], []
            for _ in range(TOP_K):
                parts = []
                for c4 in range(4):
                    be = c4 * 16
                    rm = lbuf[be, pl.ds(lg * 16, 16)]
                    ri = jnp.full((16,), be, jnp.int32)
                    for e in range(be + 1, be + 16):
                        v = lbuf[e, pl.ds(lg * 16, 16)]
                        take = v > rm
                        rm = jnp.where(take, v, rm)
                        ri = jnp.where(take, jnp.full((16,), e, jnp.int32), ri)
                    parts.append((rm, ri))
                rm, ri = parts[0]
                for v, vi in parts[1:]:
                    take = v > rm
                    rm = jnp.where(take, v, rm)
                    ri = jnp.where(take, vi, ri)
                vals.append(rm)
                idxs.append(ri)
                plsc.store_scatter(
                    lbuf, [ri, lg * 16 + lane],
                    jnp.full((16,), -1e30, jnp.float32))
            s = jnp.zeros((16,), jnp.float32)
            es = []
            for k in range(TOP_K):
                e_k = jnp.exp(vals[k] - vals[0])
                es.append(e_k)
                s = s + e_k
            rs = 1.0 / s
            for k in range(TOP_K):
                plsc.store_scatter(ewbuf, [tok, idxs[k]], es[k] * rs)
                plsc.store_scatter(idxbuf, [tok, jnp.full((16,), k, jnp.int32)],
                                   idxs[k])
            return carry2

        lax.fori_loop(0, TCH // 16, gbody, 0)
        tb = g * TCH
        pltpu.sync_copy(ewbuf, ew_hbm.at[pl.ds(tb, TCH), :])
        pltpu.sync_copy(idxbuf, idx_hbm.at[pl.ds(tb, TCH), :])
        return carry

    lax.fori_loop(0, ch_per_w, chunk_body, 0)


@jax.jit
def kernel(inputs, gate_kernel):
    n_tokens, d_model = inputs.shape
    grid = (n_tokens // BLOCK_T,)
    nch_blk = BLOCK_T // TCH
    n_chunks = n_tokens // TCH
    tok_spec = lambda w: pl.BlockSpec((BLOCK_T, w), lambda i: (i, 0))
    logits, probs, lchunks = pl.pallas_call(
        _tc_block,
        grid=grid,
        in_specs=[
            pl.BlockSpec((BLOCK_T, d_model), lambda i: (i, 0)),
            pl.BlockSpec((d_model, NUM_EXPERTS), lambda i: (0, 0)),
        ],
        out_specs=(
            tok_spec(NUM_EXPERTS),
            tok_spec(NUM_EXPERTS),
            pl.BlockSpec((nch_blk, NUM_EXPERTS, TCH), lambda i: (i, 0, 0)),
        ),
        out_shape=(
            jax.ShapeDtypeStruct((n_tokens, NUM_EXPERTS), jnp.float32),
            jax.ShapeDtypeStruct((n_tokens, NUM_EXPERTS), jnp.float32),
            jax.ShapeDtypeStruct((n_chunks, NUM_EXPERTS, TCH), jnp.float32),
        ),
        compiler_params=pltpu.CompilerParams(
            dimension_semantics=("arbitrary",),
        ),
    )(inputs, gate_kernel)

    ch_per_w = n_chunks // NW
    mesh = plsc.VectorSubcoreMesh(
        core_axis_name="c", subcore_axis_name="s",
        num_cores=NC, num_subcores=NS)
    ew, idx = pl.kernel(
        functools.partial(_sc_routing, ch_per_w=ch_per_w),
        out_type=(
            jax.ShapeDtypeStruct((n_tokens, NUM_EXPERTS), jnp.float32),
            jax.ShapeDtypeStruct((n_tokens, TOP_K), jnp.int32),
        ),
        mesh=mesh,
        compiler_params=pltpu.CompilerParams(needs_layout_passes=False),
        scratch_types=[
            pltpu.VMEM((NUM_EXPERTS, TCH), jnp.float32),
            pltpu.VMEM((TCH, NUM_EXPERTS), jnp.float32),
            pltpu.VMEM((TCH, TOP_K), jnp.int32),
        ],
    )(lchunks)
    return (ew, idx, logits, probs)
